# baseline (speedup 1.0000x reference)
import jax
import jax.numpy as jnp
from jax import lax
from jax.experimental import pallas as pl
from jax.experimental.pallas import tpu as pltpu
from jax.experimental.pallas import tpu_sc as plsc

f32 = jnp.float32
i32 = jnp.int32

NC = 2
NS = 16
NW = NC * NS
K = 80
LANES = 16
CHUNK = 8

N_PAD = 10112


def _slot(ci):
    return ci % 3


def _deg_body(row_hbm, ew_hbm, degp_hbm, idx_v, ew_v, u0, u1, acc,
              sc0, sc1, sc2, ss0, ss1):
    n_edges = row_hbm.shape[0]
    epw = n_edges // NW
    nwin = epw // K
    c = lax.axis_index("c")
    s = lax.axis_index("s")
    w = c * NS + s
    rows_per_w = N_PAD // NS
    base_r = s * rows_per_w

    zrow = jnp.zeros((1, LANES), f32)

    @pl.loop(0, K)
    def _(i):
        u0[pl.ds(i, 1), :] = zrow

    nfull = rows_per_w // K
    for t in range(nfull):
        pltpu.sync_copy(u0, acc.at[pl.ds(base_r + t * K, K)])
    rem = rows_per_w - nfull * K
    if rem:
        pltpu.sync_copy(u0.at[pl.ds(0, rem)],
                        acc.at[pl.ds(base_r + nfull * K, rem)])

    plsc.subcore_barrier()

    ebase = w * epw

    @pl.loop(0, nwin)
    def _(t):
        b = ebase + t * K
        pltpu.sync_copy(row_hbm.at[pl.ds(b, K)], idx_v)
        pltpu.sync_copy(ew_hbm.at[pl.ds(b, K)], ew_v)

        @pl.loop(0, K // LANES)
        def _(jj):
            v = ew_v[pl.ds(jj * LANES, LANES)]
            for l in range(LANES):
                u0[pl.ds(jj * LANES + l, 1), :] = jnp.full(
                    (1, LANES), v[l], f32)

        pltpu.sync_copy(u0, acc.at[idx_v], add=True)

    plsc.subcore_barrier()
    pltpu.sync_copy(acc.at[pl.ds(base_r, rows_per_w)],
                    degp_hbm.at[c, pl.ds(base_r, rows_per_w)])


def _deg_partial(row1, ew1):
    kern = pl.kernel(
        _deg_body,
        out_type=jax.ShapeDtypeStruct((NC, N_PAD, LANES), f32),
        mesh=plsc.VectorSubcoreMesh(core_axis_name="c", subcore_axis_name="s"),
        scratch_types=[
            pltpu.VMEM((K,), i32),
            pltpu.VMEM((K,), f32),
            pltpu.VMEM((K, LANES), f32),
            pltpu.VMEM((K, LANES), f32),
            pltpu.VMEM_SHARED((N_PAD, LANES), f32),
            pltpu.SemaphoreType.DMA,
            pltpu.SemaphoreType.DMA,
            pltpu.SemaphoreType.DMA,
            pltpu.SemaphoreType.DMA,
            pltpu.SemaphoreType.DMA,
        ],
    )
    return kern(row1, ew1)


def _mp_body(xs_hbm, z_hbm, row_hbm, col_hbm, ew_hbm, accp_hbm,
             col_v, row_v, ew_v, g0, bcb, acc,
             sc0, sc1, sc2, sg0, sg1, sg2, ss0, ss1, ss2):
    n_edges = row_hbm.shape[0]
    epw = n_edges // NW
    nwin = epw // K
    c = lax.axis_index("c")
    s = lax.axis_index("s")
    w = c * NS + s
    rows_per_w = N_PAD // NS
    base_r = s * rows_per_w

    @pl.when(c == 0)
    def _():
        pltpu.sync_copy(xs_hbm.at[pl.ds(base_r, rows_per_w)],
                        acc.at[pl.ds(base_r, rows_per_w)])

    @pl.when(c != 0)
    def _():
        pltpu.sync_copy(z_hbm.at[pl.ds(base_r, rows_per_w)],
                        acc.at[pl.ds(base_r, rows_per_w)])

    plsc.subcore_barrier()

    ebase = w * epw

    @pl.loop(0, nwin)
    def _(t):
        b = ebase + t * K
        pltpu.sync_copy(col_hbm.at[pl.ds(b, K)], col_v)
        pltpu.sync_copy(row_hbm.at[pl.ds(b, K)], row_v)
        pltpu.sync_copy(ew_hbm.at[pl.ds(b, K)], ew_v)
        pltpu.sync_copy(xs_hbm.at[col_v], g0)

        @pl.loop(0, K // LANES)
        def _(jj):
            v = ew_v[pl.ds(jj * LANES, LANES)]
            for l in range(LANES):
                bcb[pl.ds(jj * LANES + l, 1), :] = jnp.full(
                    (1, LANES), v[l], f32)

        @pl.loop(0, K)
        def _(j):
            bc = bcb[pl.ds(j, 1), :]
            for cc in range(8):
                sl = (pl.ds(j, 1), pl.ds(cc * LANES, LANES))
                g0[sl] = g0[sl] * bc

        pltpu.sync_copy(g0, acc.at[row_v], add=True)

    plsc.subcore_barrier()
    pltpu.sync_copy(acc.at[pl.ds(base_r, rows_per_w)],
                    accp_hbm.at[c, pl.ds(base_r, rows_per_w)])


def _mp_partial(xs, zeros_hbm, row1, col1, ew1):
    kern = pl.kernel(
        _mp_body,
        out_type=jax.ShapeDtypeStruct((NC, N_PAD, 128), f32),
        mesh=plsc.VectorSubcoreMesh(core_axis_name="c", subcore_axis_name="s"),
        scratch_types=[
            pltpu.VMEM((K,), i32),
            pltpu.VMEM((K,), i32),
            pltpu.VMEM((K,), f32),
            pltpu.VMEM((K, 128), f32),
            pltpu.VMEM((K, LANES), f32),
            pltpu.VMEM_SHARED((N_PAD, 128), f32),
            pltpu.SemaphoreType.DMA,
            pltpu.SemaphoreType.DMA,
            pltpu.SemaphoreType.DMA,
            pltpu.SemaphoreType.DMA,
            pltpu.SemaphoreType.DMA,
            pltpu.SemaphoreType.DMA,
            pltpu.SemaphoreType.DMA,
            pltpu.SemaphoreType.DMA,
            pltpu.SemaphoreType.DMA,
        ],
    )
    return kern(xs, zeros_hbm, row1, col1, ew1)


BLK = 1264


def _mm_body(x_ref, w_ref, o_ref):
    o_ref[...] = jnp.dot(x_ref[...], w_ref[...],
                         preferred_element_type=f32,
                         precision=lax.Precision.HIGHEST)


def _matmul(x_pad, W):
    return pl.pallas_call(
        _mm_body,
        grid=(N_PAD // BLK,),
        in_specs=[
            pl.BlockSpec((BLK, 128), lambda i: (i, 0)),
            pl.BlockSpec((128, 128), lambda i: (0, 0)),
        ],
        out_specs=pl.BlockSpec((BLK, 128), lambda i: (i, 0)),
        out_shape=jax.ShapeDtypeStruct((N_PAD, 128), f32),
    )(x_pad, W)


def _scale_body(degp_ref, xw_ref, xs_ref, dinvb_ref):
    deg = 1.0 + degp_ref[0, :, 0:1] + degp_ref[1, :, 0:1]
    dinv = lax.rsqrt(deg)
    xs_ref[...] = dinv * xw_ref[...]
    dinvb_ref[...] = jnp.broadcast_to(dinv, xw_ref.shape)


def _scale(degp, xw):
    return pl.pallas_call(
        _scale_body,
        grid=(N_PAD // BLK,),
        in_specs=[
            pl.BlockSpec((NC, BLK, LANES), lambda i: (0, i, 0)),
            pl.BlockSpec((BLK, 128), lambda i: (i, 0)),
        ],
        out_specs=[
            pl.BlockSpec((BLK, 128), lambda i: (i, 0)),
            pl.BlockSpec((BLK, 128), lambda i: (i, 0)),
        ],
        out_shape=[
            jax.ShapeDtypeStruct((N_PAD, 128), f32),
            jax.ShapeDtypeStruct((N_PAD, 128), f32),
        ],
    )(degp, xw)


def _fin_body(accp_ref, dinvb_ref, b_ref, o_ref):
    o_ref[...] = dinvb_ref[...] * (accp_ref[0] + accp_ref[1]) + b_ref[...]


def _finish(accp, dinvb, b2d):
    return pl.pallas_call(
        _fin_body,
        grid=(N_PAD // BLK,),
        in_specs=[
            pl.BlockSpec((NC, BLK, 128), lambda i: (0, i, 0)),
            pl.BlockSpec((BLK, 128), lambda i: (i, 0)),
            pl.BlockSpec((1, 128), lambda i: (0, 0)),
        ],
        out_specs=pl.BlockSpec((BLK, 128), lambda i: (i, 0)),
        out_shape=jax.ShapeDtypeStruct((N_PAD, 128), f32),
    )(accp, dinvb, b2d)


def kernel(x, edge_index, edge_weight, W, b):
    n = x.shape[1]
    n_edges = edge_index.shape[1]
    grp = NW * K * CHUNK
    e_tot = -(-n_edges // grp) * grp
    e_pad = e_tot - n_edges
    nwin = e_tot // (NW * K)
    pad_idx = (jnp.arange(e_pad, dtype=i32) * 37) % n
    row2 = jnp.concatenate([edge_index[0].astype(i32), pad_idx])
    col2 = jnp.concatenate([edge_index[1].astype(i32), pad_idx])
    ew2 = jnp.concatenate(
        [edge_weight.astype(f32), jnp.zeros((e_pad,), f32)])

    x_pad = jnp.pad(x[0], ((0, N_PAD - n), (0, 0)))
    zeros_hbm = jnp.zeros((N_PAD, 128), f32)

    degp = _deg_partial(row2, ew2)
    xw = _matmul(x_pad, W)
    xs, dinvb = _scale(degp, xw)
    accp = _mp_partial(xs, zeros_hbm, row2, col2, ew2)
    out = _finish(accp, dinvb, b.reshape(1, 128))

    return out[:n].reshape(1, n, -1)


# speedup vs baseline: 2.6094x; 2.6094x over previous
import jax
import jax.numpy as jnp
from jax import lax
from jax.experimental import pallas as pl
from jax.experimental.pallas import tpu as pltpu
from jax.experimental.pallas import tpu_sc as plsc

f32 = jnp.float32
i32 = jnp.int32

NC = 2
NS = 16
NW = NC * NS
K = 80
LANES = 16
CHUNK = 8

N_PAD = 10112


def _slot(ci):
    return ci % 3


def _deg_body(row_hbm, ew_hbm, degp_hbm, rv0, rv1, rv2, rv3,
              ev0, ev1, ev2, ev3, u0, u1, acc,
              si0, si1, si2, si3, ss0, ss1):
    n_edges = row_hbm.shape[0]
    epw = n_edges // NW
    nwin = epw // K
    c = lax.axis_index("c")
    s = lax.axis_index("s")
    w = c * NS + s
    rows_per_w = N_PAD // NS
    base_r = s * rows_per_w
    ebase = w * epw

    isems = (si0, si1, si2, si3)
    rowvs = (rv0, rv1, rv2, rv3)
    ewvs = (ev0, ev1, ev2, ev3)
    ubufs = (u0, u1)
    ssems = (ss0, ss1)

    def idx_refs(t, r):
        src = pl.ds(ebase + t * K, K)
        return ((row_hbm.at[src], rowvs[r]),
                (ew_hbm.at[src], ewvs[r]))

    def idx_load(t, r):
        for sref, dref in idx_refs(t, r):
            pltpu.async_copy(sref, dref, isems[r])

    def idx_wait(t, r):
        for sref, dref in idx_refs(t, r):
            pltpu.make_async_copy(sref, dref, isems[r]).wait()

    for t0 in range(4):
        idx_load(t0, t0)

    zrow = jnp.zeros((1, LANES), f32)

    @pl.loop(0, K)
    def _(i):
        u0[pl.ds(i, 1), :] = zrow

    nfull = rows_per_w // K
    for t in range(nfull):
        pltpu.sync_copy(u0, acc.at[pl.ds(base_r + t * K, K)])
    rem = rows_per_w - nfull * K
    if rem:
        pltpu.sync_copy(u0.at[pl.ds(0, rem)],
                        acc.at[pl.ds(base_r + nfull * K, rem)])

    idx_wait(0, 0)
    idx_wait(1, 1)
    plsc.subcore_barrier()

    def build(r, bi):
        u = ubufs[bi]

        @pl.loop(0, K // LANES)
        def _(jj):
            v = ewvs[r][pl.ds(jj * LANES, LANES)]
            for l in range(LANES):
                u[pl.ds(jj * LANES + l, 1), :] = jnp.full(
                    (1, LANES), v[l], f32)

    def scat(r, bi):
        pltpu.async_copy(ubufs[bi], acc.at[rowvs[r]], ssems[bi], add=True)

    def scat_wait(r, bi):
        pltpu.make_async_copy(ubufs[bi], acc.at[rowvs[r]],
                              ssems[bi]).wait()

    for t in range(CHUNK):
        if t >= 2:
            scat_wait((t - 2) % 4, (t - 2) % 2)
            idx_load(t + 2, (t + 2) % 4)
        if t >= 1:
            idx_wait(t + 1, (t + 1) % 4)
        build(t % 4, t % 2)
        scat(t % 4, t % 2)

    nv = (nwin - CHUNK) // (3 * CHUNK)

    @pl.loop(0, nv)
    def _(v):
        tb = 3 * CHUNK * v + CHUNK
        for k in range(3 * CHUNK):
            t = tb + k
            scat_wait((k + 2) % 4, k % 2)
            if k < 3 * CHUNK - 2:
                idx_load(t + 2, (k + 2) % 4)
            else:

                @pl.when(v < nv - 1)
                def _():
                    idx_load(t + 2, (k + 2) % 4)
            if k < 3 * CHUNK - 1:
                idx_wait(t + 1, (k + 1) % 4)
            else:

                @pl.when(v < nv - 1)
                def _():
                    idx_wait(t + 1, (k + 1) % 4)
            build(k % 4, k % 2)
            scat(k % 4, k % 2)

    scat_wait((nwin - 2) % 4, (nwin - 2) % 2)
    scat_wait((nwin - 1) % 4, (nwin - 1) % 2)

    plsc.subcore_barrier()
    pltpu.sync_copy(acc.at[pl.ds(base_r, rows_per_w)],
                    degp_hbm.at[c, pl.ds(base_r, rows_per_w)])


def _deg_partial(row1, ew1):
    kern = pl.kernel(
        _deg_body,
        out_type=jax.ShapeDtypeStruct((NC, N_PAD, LANES), f32),
        mesh=plsc.VectorSubcoreMesh(core_axis_name="c", subcore_axis_name="s"),
        scratch_types=[
            pltpu.VMEM((K,), i32),
            pltpu.VMEM((K,), i32),
            pltpu.VMEM((K,), i32),
            pltpu.VMEM((K,), i32),
            pltpu.VMEM((K,), f32),
            pltpu.VMEM((K,), f32),
            pltpu.VMEM((K,), f32),
            pltpu.VMEM((K,), f32),
            pltpu.VMEM((K, LANES), f32),
            pltpu.VMEM((K, LANES), f32),
            pltpu.VMEM_SHARED((N_PAD, LANES), f32),
            pltpu.SemaphoreType.DMA,
            pltpu.SemaphoreType.DMA,
            pltpu.SemaphoreType.DMA,
            pltpu.SemaphoreType.DMA,
            pltpu.SemaphoreType.DMA,
            pltpu.SemaphoreType.DMA,
        ],
    )
    return kern(row1, ew1)


def _mp_body(xs_hbm, z_hbm, row_hbm, col_hbm, ew_hbm, accp_hbm,
             cv0, cv1, cv2, cv3, rv0, rv1, rv2, rv3,
             ev0, ev1, ev2, ev3, g0, g1, g2, bcb, acc,
             si0, si1, si2, si3, sg0, sg1, sg2, ss0, ss1, ss2):
    n_edges = row_hbm.shape[0]
    epw = n_edges // NW
    nwin = epw // K
    c = lax.axis_index("c")
    s = lax.axis_index("s")
    w = c * NS + s
    rows_per_w = N_PAD // NS
    base_r = s * rows_per_w
    ebase = w * epw

    isems = (si0, si1, si2, si3)
    colvs = (cv0, cv1, cv2, cv3)
    rowvs = (rv0, rv1, rv2, rv3)
    ewvs = (ev0, ev1, ev2, ev3)
    bufs = (g0, g1, g2)
    gsems = (sg0, sg1, sg2)
    ssems = (ss0, ss1, ss2)

    def idx_refs(t, r):
        src = pl.ds(ebase + t * K, K)
        return ((col_hbm.at[src], colvs[r]),
                (row_hbm.at[src], rowvs[r]),
                (ew_hbm.at[src], ewvs[r]))

    def idx_load(t, r):
        for sref, dref in idx_refs(t, r):
            pltpu.async_copy(sref, dref, isems[r])

    def idx_wait(t, r):
        for sref, dref in idx_refs(t, r):
            pltpu.make_async_copy(sref, dref, isems[r]).wait()

    for t0 in range(4):
        idx_load(t0, t0)

    @pl.when(c == 0)
    def _():
        pltpu.sync_copy(xs_hbm.at[pl.ds(base_r, rows_per_w)],
                        acc.at[pl.ds(base_r, rows_per_w)])

    @pl.when(c != 0)
    def _():
        pltpu.sync_copy(z_hbm.at[pl.ds(base_r, rows_per_w)],
                        acc.at[pl.ds(base_r, rows_per_w)])

    idx_wait(0, 0)
    idx_wait(1, 1)
    plsc.subcore_barrier()

    def gather(r, bi):
        pltpu.async_copy(xs_hbm.at[colvs[r]], bufs[bi], gsems[bi])

    def gather_wait(r, bi):
        pltpu.make_async_copy(xs_hbm.at[colvs[r]], bufs[bi],
                              gsems[bi]).wait()

    def scat(r, bi):
        pltpu.async_copy(bufs[bi], acc.at[rowvs[r]], ssems[bi], add=True)

    def scat_wait(r, bi):
        pltpu.make_async_copy(bufs[bi], acc.at[rowvs[r]],
                              ssems[bi]).wait()

    def scale(r, bi):
        g = bufs[bi]

        @pl.loop(0, K // LANES)
        def _(jj):
            v = ewvs[r][pl.ds(jj * LANES, LANES)]
            for l in range(LANES):
                bcb[pl.ds(jj * LANES + l, 1), :] = jnp.full(
                    (1, LANES), v[l], f32)

        @pl.loop(0, K)
        def _(j):
            bc = bcb[pl.ds(j, 1), :]
            for cc in range(8):
                sl = (pl.ds(j, 1), pl.ds(cc * LANES, LANES))
                g[sl] = g[sl] * bc

    gather(0, 0)
    gather(1, 1)

    for t in range(CHUNK):
        if t >= 2:
            scat_wait((t - 2) % 4, (t - 2) % 3)
            idx_load(t + 2, (t + 2) % 4)
        if t >= 1:
            idx_wait(t + 1, (t + 1) % 4)
            gather((t + 1) % 4, (t + 1) % 3)
        gather_wait(t % 4, t % 3)
        scale(t % 4, t % 3)
        scat(t % 4, t % 3)

    nv = (nwin - CHUNK) // (3 * CHUNK)

    @pl.loop(0, nv)
    def _(v):
        tb = 3 * CHUNK * v + CHUNK
        for k in range(3 * CHUNK):
            t = tb + k
            rc = k % 4
            bc_ = (2 + k) % 3
            scat_wait((k + 2) % 4, k % 3)
            if k < 3 * CHUNK - 2:
                idx_load(t + 2, (k + 2) % 4)
            else:

                @pl.when(v < nv - 1)
                def _():
                    idx_load(t + 2, (k + 2) % 4)
            if k < 3 * CHUNK - 1:
                idx_wait(t + 1, (k + 1) % 4)
                gather((k + 1) % 4, k % 3)
            else:

                @pl.when(v < nv - 1)
                def _():
                    idx_wait(t + 1, (k + 1) % 4)
                    gather((k + 1) % 4, k % 3)
            gather_wait(rc, bc_)
            scale(rc, bc_)
            scat(rc, bc_)

    scat_wait((nwin - 2) % 4, (nwin - 2) % 3)
    scat_wait((nwin - 1) % 4, (nwin - 1) % 3)

    plsc.subcore_barrier()
    pltpu.sync_copy(acc.at[pl.ds(base_r, rows_per_w)],
                    accp_hbm.at[c, pl.ds(base_r, rows_per_w)])


def _mp_partial(xs, zeros_hbm, row1, col1, ew1):
    kern = pl.kernel(
        _mp_body,
        out_type=jax.ShapeDtypeStruct((NC, N_PAD, 128), f32),
        mesh=plsc.VectorSubcoreMesh(core_axis_name="c", subcore_axis_name="s"),
        scratch_types=[
            pltpu.VMEM((K,), i32),
            pltpu.VMEM((K,), i32),
            pltpu.VMEM((K,), i32),
            pltpu.VMEM((K,), i32),
            pltpu.VMEM((K,), i32),
            pltpu.VMEM((K,), i32),
            pltpu.VMEM((K,), i32),
            pltpu.VMEM((K,), i32),
            pltpu.VMEM((K,), f32),
            pltpu.VMEM((K,), f32),
            pltpu.VMEM((K,), f32),
            pltpu.VMEM((K,), f32),
            pltpu.VMEM((K, 128), f32),
            pltpu.VMEM((K, 128), f32),
            pltpu.VMEM((K, 128), f32),
            pltpu.VMEM((K, LANES), f32),
            pltpu.VMEM_SHARED((N_PAD, 128), f32),
            pltpu.SemaphoreType.DMA,
            pltpu.SemaphoreType.DMA,
            pltpu.SemaphoreType.DMA,
            pltpu.SemaphoreType.DMA,
            pltpu.SemaphoreType.DMA,
            pltpu.SemaphoreType.DMA,
            pltpu.SemaphoreType.DMA,
            pltpu.SemaphoreType.DMA,
            pltpu.SemaphoreType.DMA,
            pltpu.SemaphoreType.DMA,
        ],
    )
    return kern(xs, zeros_hbm, row1, col1, ew1)


BLK = 1264


def _mm_body(x_ref, w_ref, o_ref):
    o_ref[...] = jnp.dot(x_ref[...], w_ref[...],
                         preferred_element_type=f32,
                         precision=lax.Precision.HIGHEST)


def _matmul(x_pad, W):
    return pl.pallas_call(
        _mm_body,
        grid=(N_PAD // BLK,),
        in_specs=[
            pl.BlockSpec((BLK, 128), lambda i: (i, 0)),
            pl.BlockSpec((128, 128), lambda i: (0, 0)),
        ],
        out_specs=pl.BlockSpec((BLK, 128), lambda i: (i, 0)),
        out_shape=jax.ShapeDtypeStruct((N_PAD, 128), f32),
    )(x_pad, W)


def _scale_body(degp_ref, xw_ref, xs_ref, dinvb_ref):
    deg = 1.0 + degp_ref[0, :, 0:1] + degp_ref[1, :, 0:1]
    dinv = lax.rsqrt(deg)
    xs_ref[...] = dinv * xw_ref[...]
    dinvb_ref[...] = jnp.broadcast_to(dinv, xw_ref.shape)


def _scale(degp, xw):
    return pl.pallas_call(
        _scale_body,
        grid=(N_PAD // BLK,),
        in_specs=[
            pl.BlockSpec((NC, BLK, LANES), lambda i: (0, i, 0)),
            pl.BlockSpec((BLK, 128), lambda i: (i, 0)),
        ],
        out_specs=[
            pl.BlockSpec((BLK, 128), lambda i: (i, 0)),
            pl.BlockSpec((BLK, 128), lambda i: (i, 0)),
        ],
        out_shape=[
            jax.ShapeDtypeStruct((N_PAD, 128), f32),
            jax.ShapeDtypeStruct((N_PAD, 128), f32),
        ],
    )(degp, xw)


def _fin_body(accp_ref, dinvb_ref, b_ref, o_ref):
    o_ref[...] = dinvb_ref[...] * (accp_ref[0] + accp_ref[1]) + b_ref[...]


def _finish(accp, dinvb, b2d):
    return pl.pallas_call(
        _fin_body,
        grid=(N_PAD // BLK,),
        in_specs=[
            pl.BlockSpec((NC, BLK, 128), lambda i: (0, i, 0)),
            pl.BlockSpec((BLK, 128), lambda i: (i, 0)),
            pl.BlockSpec((1, 128), lambda i: (0, 0)),
        ],
        out_specs=pl.BlockSpec((BLK, 128), lambda i: (i, 0)),
        out_shape=jax.ShapeDtypeStruct((N_PAD, 128), f32),
    )(accp, dinvb, b2d)


def kernel(x, edge_index, edge_weight, W, b):
    n = x.shape[1]
    n_edges = edge_index.shape[1]
    grp = NW * K * CHUNK
    e_tot = -(-n_edges // grp) * grp
    e_pad = e_tot - n_edges
    nwin = e_tot // (NW * K)
    pad_idx = (jnp.arange(e_pad, dtype=i32) * 37) % n
    row2 = jnp.concatenate([edge_index[0].astype(i32), pad_idx])
    col2 = jnp.concatenate([edge_index[1].astype(i32), pad_idx])
    ew2 = jnp.concatenate(
        [edge_weight.astype(f32), jnp.zeros((e_pad,), f32)])

    x_pad = jnp.pad(x[0], ((0, N_PAD - n), (0, 0)))
    zeros_hbm = jnp.zeros((N_PAD, 128), f32)

    degp = _deg_partial(row2, ew2)
    xw = _matmul(x_pad, W)
    xs, dinvb = _scale(degp, xw)
    accp = _mp_partial(xs, zeros_hbm, row2, col2, ew2)
    out = _finish(accp, dinvb, b.reshape(1, 128))

    return out[:n].reshape(1, n, -1)
